# baseline (speedup 1.0000x reference)
import functools

import jax
import jax.numpy as jnp
from jax import lax
from jax.experimental import pallas as pl
from jax.experimental.pallas import tpu as pltpu
from jax.experimental.pallas import tpu_sc as plsc

VOCAB = 1000000
EMBED_DIM = 64
NUM_CLASS = 1000
BATCH = 4096
SEQ = 200

NUM_CORES = 2
NUM_SUBCORES = 16
NUM_WORKERS = NUM_CORES * NUM_SUBCORES
B_PER_W = BATCH // NUM_WORKERS
ROW = 128
S0 = 128
S1 = SEQ - S0

NBUF = 2
UNROLL = 8

NBLK = VOCAB // 128
TAIL = VOCAB - NBLK * 128


def _transpose_body(tt_hbm, tail_hbm, out_hbm, in_blk, out_blk, tail_blk,
                    sin, sout):
    wid = lax.axis_index("c") * NUM_SUBCORES + lax.axis_index("s")

    d_idx = [jnp.arange(16, dtype=jnp.int32) + 16 * k for k in range(4)]

    def start_in(j, b):
        pltpu.async_copy(
            tt_hbm.at[:, pl.ds(j * 128, 128)], in_blk.at[b], sin.at[b])

    def wait_in(b):
        pltpu.make_async_copy(
            tt_hbm.at[:, pl.ds(0, 128)], in_blk.at[b], sin.at[b]).wait()

    def start_out(j, b):
        pltpu.async_copy(
            out_blk.at[b], out_hbm.at[pl.ds(j * 128, 128), :], sout.at[b])

    def wait_out(b):
        pltpu.make_async_copy(
            out_blk.at[b], out_hbm.at[pl.ds(0, 128), :], sout.at[b]).wait()

    def compute_block(b):
        def row_body(r, carry):
            c_idx = jnp.full((16,), r, dtype=jnp.int32)
            for k in range(4):
                x = plsc.load_gather(in_blk.at[b], [d_idx[k], c_idx])
                out_blk[b, r, pl.ds(16 * k, 16)] = x
            return carry
        lax.fori_loop(0, 128, row_body, 0)

    @pl.when(wid < NBLK)
    def _():
        start_in(wid, 0)

    @pl.when(wid + 32 < NBLK)
    def _():
        start_in(wid + 32, 1)

    def outer(i, carry):
        for b in range(2):
            j = wid + 32 * (2 * i + b)

            @pl.when(j < NBLK)
            def _():
                wait_in(b)

                @pl.when(2 * i + b >= 2)
                def _():
                    wait_out(b)

                compute_block(b)
                start_out(j, b)

                @pl.when(j + 64 < NBLK)
                def _():
                    start_in(j + 64, b)
        return carry

    lax.fori_loop(0, (NBLK + 63) // 64, outer, 0)
    wait_out(0)
    wait_out(1)

    @pl.when(wid == 0)
    def _():
        pltpu.sync_copy(tail_hbm, tail_blk)

        def tail_row(r, carry):
            c_idx = jnp.full((16,), r, dtype=jnp.int32)
            for k in range(4):
                x = plsc.load_gather(tail_blk, [d_idx[k], c_idx])
                out_blk[0, r, pl.ds(16 * k, 16)] = x
            return carry

        lax.fori_loop(0, TAIL, tail_row, 0)
        pltpu.sync_copy(
            out_blk.at[0, pl.ds(0, TAIL)],
            out_hbm.at[pl.ds(NBLK * 128, TAIL), :])


def _sc_transpose(tableT, tail64):
    mesh = plsc.VectorSubcoreMesh(core_axis_name="c", subcore_axis_name="s")
    f = pl.kernel(
        _transpose_body,
        out_type=jax.ShapeDtypeStruct((VOCAB, ROW), jnp.float32),
        mesh=mesh,
        scratch_types=[
            pltpu.VMEM((2, EMBED_DIM, 128), jnp.float32),
            pltpu.VMEM((2, 128, ROW), jnp.float32),
            pltpu.VMEM((EMBED_DIM, TAIL), jnp.float32),
            pltpu.SemaphoreType.DMA((2,)),
            pltpu.SemaphoreType.DMA((2,)),
        ],
        compiler_params=pltpu.CompilerParams(
            use_tc_tiling_on_sc=True, needs_layout_passes=False),
    )
    return f(tableT, tail64)


def _pool_body(ids_hbm, table_hbm, out_hbm, idx_v, gbuf, pooled_v, sems):
    wid = lax.axis_index("c") * NUM_SUBCORES + lax.axis_index("s")
    base = wid * B_PER_W
    pltpu.sync_copy(ids_hbm.at[pl.ds(base, B_PER_W), :], idx_v)

    inv_seq = jnp.float32(1.0 / SEQ)

    def start_gather(r, b):
        pltpu.async_copy(
            table_hbm.at[idx_v.at[r, pl.ds(0, S0)]],
            gbuf.at[b, pl.ds(0, S0)], sems.at[b])
        pltpu.async_copy(
            table_hbm.at[idx_v.at[r, pl.ds(S0, S1)]],
            gbuf.at[b, pl.ds(S0, S1)], sems.at[b])

    def wait_gather(b):
        pltpu.make_async_copy(
            table_hbm.at[idx_v.at[0, pl.ds(0, S0)]],
            gbuf.at[b, pl.ds(0, S0)], sems.at[b]).wait()
        pltpu.make_async_copy(
            table_hbm.at[idx_v.at[0, pl.ds(S0, S1)]],
            gbuf.at[b, pl.ds(S0, S1)], sems.at[b]).wait()

    def reduce_slot(r, b):
        def red_body(j, accs):
            accs = list(accs)
            for u in range(UNROLL):
                row = j * UNROLL + u
                for k in range(4):
                    a = u % 2 + 2 * k
                    accs[a] = accs[a] + gbuf[b, row, pl.ds(16 * k, 16)]
            return tuple(accs)

        zero = jnp.zeros((16,), jnp.float32)
        accs = lax.fori_loop(0, SEQ // UNROLL, red_body, (zero,) * 8)
        for k in range(4):
            pooled_v[r, pl.ds(16 * k, 16)] = (
                (accs[2 * k] + accs[2 * k + 1]) * inv_seq)

    for b in range(NBUF):
        start_gather(b, b)

    def outer(g, carry):
        for b in range(NBUF):
            r = g * NBUF + b
            wait_gather(b)
            reduce_slot(r, b)

            @pl.when(r + NBUF < B_PER_W)
            def _():
                start_gather(r + NBUF, b)
        return carry

    lax.fori_loop(0, B_PER_W // NBUF, outer, 0)
    pltpu.sync_copy(pooled_v, out_hbm.at[pl.ds(base, B_PER_W), :])


def _sc_pool(input_ids, table128):
    mesh = plsc.VectorSubcoreMesh(core_axis_name="c", subcore_axis_name="s")
    f = pl.kernel(
        _pool_body,
        out_type=jax.ShapeDtypeStruct((BATCH, EMBED_DIM), jnp.float32),
        mesh=mesh,
        scratch_types=[
            pltpu.VMEM((B_PER_W, SEQ), jnp.int32),
            pltpu.VMEM((NBUF, SEQ, ROW), jnp.float32),
            pltpu.VMEM((B_PER_W, EMBED_DIM), jnp.float32),
            pltpu.SemaphoreType.DMA((NBUF,)),
        ],
        compiler_params=pltpu.CompilerParams(use_tc_tiling_on_sc=False),
    )
    return f(input_ids, table128)


BM = 256


def _matmul_body(p_ref, w_ref, b_ref, o_ref):
    acc = lax.dot_general(
        p_ref[...], w_ref[...],
        dimension_numbers=(((1,), (1,)), ((), ())),
        preferred_element_type=jnp.float32)
    o_ref[...] = acc + b_ref[...]


def _tc_head(pooled, fc_w, fc_b):
    bias = fc_b.reshape(1, NUM_CLASS)
    return pl.pallas_call(
        _matmul_body,
        grid=(BATCH // BM,),
        in_specs=[
            pl.BlockSpec((BM, EMBED_DIM), lambda i: (i, 0)),
            pl.BlockSpec((NUM_CLASS, EMBED_DIM), lambda i: (0, 0)),
            pl.BlockSpec((1, NUM_CLASS), lambda i: (0, 0)),
        ],
        out_specs=pl.BlockSpec((BM, NUM_CLASS), lambda i: (i, 0)),
        out_shape=jax.ShapeDtypeStruct((BATCH, NUM_CLASS), jnp.float32),
    )(pooled, fc_w, bias)


def kernel(input_ids, emb_table, fc_w, fc_b):
    tableT = emb_table.T
    tail64 = emb_table[NBLK * 128:].T
    table128 = _sc_transpose(tableT, tail64)
    pooled = _sc_pool(input_ids, table128)
    return _tc_head(pooled, fc_w, fc_b)


# speedup vs baseline: 2.9573x; 2.9573x over previous
import functools

import jax
import jax.numpy as jnp
from jax import lax
from jax.experimental import pallas as pl
from jax.experimental.pallas import tpu as pltpu
from jax.experimental.pallas import tpu_sc as plsc

VOCAB = 1000000
EMBED_DIM = 64
NUM_CLASS = 1000
BATCH = 4096
SEQ = 200

NUM_CORES = 2
NUM_SUBCORES = 16
NUM_WORKERS = NUM_CORES * NUM_SUBCORES
B_PER_W = BATCH // NUM_WORKERS
ROW = 128
S0 = 128
S1 = SEQ - S0

NBUF = 2
UNROLL = 8

TBLK = 4096


def _transpose_tc_body(t_ref, o_ref):
    x = t_ref[...]
    xt = x.T
    o_ref[...] = jnp.concatenate([xt, jnp.zeros_like(xt)], axis=1)


def _tc_transpose(tableT):
    grid = (VOCAB + TBLK - 1) // TBLK
    return pl.pallas_call(
        _transpose_tc_body,
        grid=(grid,),
        in_specs=[pl.BlockSpec((EMBED_DIM, TBLK), lambda i: (0, i))],
        out_specs=pl.BlockSpec((TBLK, ROW), lambda i: (i, 0)),
        out_shape=jax.ShapeDtypeStruct((VOCAB, ROW), jnp.float32),
    )(tableT)


def _pool_body(ids_hbm, table_hbm, out_hbm, idx_v, gbuf, pooled_v, sems):
    wid = lax.axis_index("c") * NUM_SUBCORES + lax.axis_index("s")
    base = wid * B_PER_W
    pltpu.sync_copy(ids_hbm.at[pl.ds(base, B_PER_W), :], idx_v)

    inv_seq = jnp.float32(1.0 / SEQ)

    def start_gather(r, b):
        pltpu.async_copy(
            table_hbm.at[idx_v.at[r, pl.ds(0, S0)]],
            gbuf.at[b, pl.ds(0, S0)], sems.at[b])
        pltpu.async_copy(
            table_hbm.at[idx_v.at[r, pl.ds(S0, S1)]],
            gbuf.at[b, pl.ds(S0, S1)], sems.at[b])

    def wait_gather(b):
        pltpu.make_async_copy(
            table_hbm.at[idx_v.at[0, pl.ds(0, S0)]],
            gbuf.at[b, pl.ds(0, S0)], sems.at[b]).wait()
        pltpu.make_async_copy(
            table_hbm.at[idx_v.at[0, pl.ds(S0, S1)]],
            gbuf.at[b, pl.ds(S0, S1)], sems.at[b]).wait()

    def reduce_slot(r, b):
        def red_body(j, accs):
            accs = list(accs)
            for u in range(UNROLL):
                row = j * UNROLL + u
                for k in range(4):
                    a = u % 2 + 2 * k
                    accs[a] = accs[a] + gbuf[b, row, pl.ds(16 * k, 16)]
            return tuple(accs)

        zero = jnp.zeros((16,), jnp.float32)
        accs = lax.fori_loop(0, SEQ // UNROLL, red_body, (zero,) * 8)
        for k in range(4):
            pooled_v[r, pl.ds(16 * k, 16)] = (
                (accs[2 * k] + accs[2 * k + 1]) * inv_seq)

    for b in range(NBUF):
        start_gather(b, b)

    def outer(g, carry):
        for b in range(NBUF):
            r = g * NBUF + b
            wait_gather(b)
            reduce_slot(r, b)

            @pl.when(r + NBUF < B_PER_W)
            def _():
                start_gather(r + NBUF, b)
        return carry

    lax.fori_loop(0, B_PER_W // NBUF, outer, 0)
    pltpu.sync_copy(pooled_v, out_hbm.at[pl.ds(base, B_PER_W), :])


def _sc_pool(input_ids, table128):
    mesh = plsc.VectorSubcoreMesh(core_axis_name="c", subcore_axis_name="s")
    f = pl.kernel(
        _pool_body,
        out_type=jax.ShapeDtypeStruct((BATCH, EMBED_DIM), jnp.float32),
        mesh=mesh,
        scratch_types=[
            pltpu.VMEM((B_PER_W, SEQ), jnp.int32),
            pltpu.VMEM((NBUF, SEQ, ROW), jnp.float32),
            pltpu.VMEM((B_PER_W, EMBED_DIM), jnp.float32),
            pltpu.SemaphoreType.DMA((NBUF,)),
        ],
        compiler_params=pltpu.CompilerParams(use_tc_tiling_on_sc=False),
    )
    return f(input_ids, table128)


BM = 256


def _matmul_body(p_ref, w_ref, b_ref, o_ref):
    acc = lax.dot_general(
        p_ref[...], w_ref[...],
        dimension_numbers=(((1,), (1,)), ((), ())),
        preferred_element_type=jnp.float32)
    o_ref[...] = acc + b_ref[...]


def _tc_head(pooled, fc_w, fc_b):
    bias = fc_b.reshape(1, NUM_CLASS)
    return pl.pallas_call(
        _matmul_body,
        grid=(BATCH // BM,),
        in_specs=[
            pl.BlockSpec((BM, EMBED_DIM), lambda i: (i, 0)),
            pl.BlockSpec((NUM_CLASS, EMBED_DIM), lambda i: (0, 0)),
            pl.BlockSpec((1, NUM_CLASS), lambda i: (0, 0)),
        ],
        out_specs=pl.BlockSpec((BM, NUM_CLASS), lambda i: (i, 0)),
        out_shape=jax.ShapeDtypeStruct((BATCH, NUM_CLASS), jnp.float32),
    )(pooled, fc_w, bias)


def kernel(input_ids, emb_table, fc_w, fc_b):
    tableT = emb_table.T
    table128 = _tc_transpose(tableT)
    pooled = _sc_pool(input_ids, table128)
    return _tc_head(pooled, fc_w, fc_b)


# speedup vs baseline: 3.6916x; 1.2483x over previous
import functools

import jax
import jax.numpy as jnp
from jax import lax
from jax.experimental import pallas as pl
from jax.experimental.pallas import tpu as pltpu
from jax.experimental.pallas import tpu_sc as plsc

VOCAB = 1000000
EMBED_DIM = 64
NUM_CLASS = 1000
BATCH = 4096
SEQ = 200

NUM_CORES = 2
NUM_SUBCORES = 16
NUM_WORKERS = NUM_CORES * NUM_SUBCORES
B_PER_W = BATCH // NUM_WORKERS
ROW = 128
S0 = 128
S1 = SEQ - S0

NBUF = 4
UNROLL = 8

TBLK = 4096


def _transpose_tc_body(t_ref, o_ref):
    x = t_ref[...]
    xt = x.T
    o_ref[...] = jnp.concatenate([xt, jnp.zeros_like(xt)], axis=1)


def _tc_transpose(tableT):
    grid = (VOCAB + TBLK - 1) // TBLK
    return pl.pallas_call(
        _transpose_tc_body,
        grid=(grid,),
        in_specs=[pl.BlockSpec((EMBED_DIM, TBLK), lambda i: (0, i))],
        out_specs=pl.BlockSpec((TBLK, ROW), lambda i: (i, 0)),
        out_shape=jax.ShapeDtypeStruct((VOCAB, ROW), jnp.float32),
    )(tableT)


def _pool_body(ids_hbm, table_hbm, out_hbm, idx_v, gbuf, pooled_v, sems):
    wid = lax.axis_index("c") * NUM_SUBCORES + lax.axis_index("s")
    base = wid * B_PER_W
    pltpu.sync_copy(ids_hbm.at[pl.ds(base, B_PER_W), :], idx_v)

    inv_seq = jnp.float32(1.0 / SEQ)

    def start_gather(r, b):
        pltpu.async_copy(
            table_hbm.at[idx_v.at[r, pl.ds(0, S0)]],
            gbuf.at[b, pl.ds(0, S0)], sems.at[b])
        pltpu.async_copy(
            table_hbm.at[idx_v.at[r, pl.ds(S0, S1)]],
            gbuf.at[b, pl.ds(S0, S1)], sems.at[b])

    def wait_gather(b):
        pltpu.make_async_copy(
            table_hbm.at[idx_v.at[0, pl.ds(0, S0)]],
            gbuf.at[b, pl.ds(0, S0)], sems.at[b]).wait()
        pltpu.make_async_copy(
            table_hbm.at[idx_v.at[0, pl.ds(S0, S1)]],
            gbuf.at[b, pl.ds(S0, S1)], sems.at[b]).wait()

    def reduce_slot(r, b):
        def red_body(j, accs):
            accs = list(accs)
            for u in range(UNROLL):
                row = j * UNROLL + u
                for k in range(4):
                    a = u % 2 + 2 * k
                    accs[a] = accs[a] + gbuf[b, row, pl.ds(16 * k, 16)]
            return tuple(accs)

        zero = jnp.zeros((16,), jnp.float32)
        accs = lax.fori_loop(0, SEQ // UNROLL, red_body, (zero,) * 8)
        for k in range(4):
            pooled_v[r, pl.ds(16 * k, 16)] = (
                (accs[2 * k] + accs[2 * k + 1]) * inv_seq)

    for b in range(NBUF):
        start_gather(b, b)

    def outer(g, carry):
        for b in range(NBUF):
            r = g * NBUF + b
            wait_gather(b)
            reduce_slot(r, b)

            @pl.when(r + NBUF < B_PER_W)
            def _():
                start_gather(r + NBUF, b)
        return carry

    lax.fori_loop(0, B_PER_W // NBUF, outer, 0)
    pltpu.sync_copy(pooled_v, out_hbm.at[pl.ds(base, B_PER_W), :])


def _sc_pool(input_ids, table128):
    mesh = plsc.VectorSubcoreMesh(core_axis_name="c", subcore_axis_name="s")
    f = pl.kernel(
        _pool_body,
        out_type=jax.ShapeDtypeStruct((BATCH, EMBED_DIM), jnp.float32),
        mesh=mesh,
        scratch_types=[
            pltpu.VMEM((B_PER_W, SEQ), jnp.int32),
            pltpu.VMEM((NBUF, SEQ, EMBED_DIM), jnp.float32),
            pltpu.VMEM((B_PER_W, EMBED_DIM), jnp.float32),
            pltpu.SemaphoreType.DMA((NBUF,)),
        ],
        compiler_params=pltpu.CompilerParams(use_tc_tiling_on_sc=False),
    )
    return f(input_ids, table128)


BM = 256


def _matmul_body(p_ref, w_ref, b_ref, o_ref):
    acc = lax.dot_general(
        p_ref[...], w_ref[...],
        dimension_numbers=(((1,), (1,)), ((), ())),
        preferred_element_type=jnp.float32)
    o_ref[...] = acc + b_ref[...]


def _tc_head(pooled, fc_w, fc_b):
    bias = fc_b.reshape(1, NUM_CLASS)
    return pl.pallas_call(
        _matmul_body,
        grid=(BATCH // BM,),
        in_specs=[
            pl.BlockSpec((BM, EMBED_DIM), lambda i: (i, 0)),
            pl.BlockSpec((NUM_CLASS, EMBED_DIM), lambda i: (0, 0)),
            pl.BlockSpec((1, NUM_CLASS), lambda i: (0, 0)),
        ],
        out_specs=pl.BlockSpec((BM, NUM_CLASS), lambda i: (i, 0)),
        out_shape=jax.ShapeDtypeStruct((BATCH, NUM_CLASS), jnp.float32),
    )(pooled, fc_w, bias)


def kernel(input_ids, emb_table, fc_w, fc_b):
    tableT = emb_table.T
    table128 = _tc_transpose(tableT)
    table2m = table128.reshape(2 * VOCAB, EMBED_DIM)
    pooled = _sc_pool(input_ids * 2, table2m)
    return _tc_head(pooled, fc_w, fc_b)


# speedup vs baseline: 3.8007x; 1.0295x over previous
import functools

import jax
import jax.numpy as jnp
from jax import lax
from jax.experimental import pallas as pl
from jax.experimental.pallas import tpu as pltpu
from jax.experimental.pallas import tpu_sc as plsc

VOCAB = 1000000
EMBED_DIM = 64
NUM_CLASS = 1000
BATCH = 4096
SEQ = 200

NUM_CORES = 2
NUM_SUBCORES = 16
NUM_WORKERS = NUM_CORES * NUM_SUBCORES
B_PER_W = BATCH // NUM_WORKERS
ROW = 128
S0 = 128
S1 = SEQ - S0

NBUF = 4
UNROLL = 8

TBLK = 2048
TGRID = 245
HPAD = TBLK * TGRID


def _transpose_tc_body(tl_ref, tr_ref, o_ref):
    o_ref[...] = jnp.concatenate(
        [tl_ref[...].T, tr_ref[...].T], axis=1)


def _tc_transpose(tableT):
    return pl.pallas_call(
        _transpose_tc_body,
        grid=(TGRID,),
        in_specs=[
            pl.BlockSpec((EMBED_DIM, TBLK), lambda i: (0, i)),
            pl.BlockSpec(
                (EMBED_DIM, TBLK),
                lambda i: (0, jnp.minimum(i + TGRID, VOCAB // TBLK))),
        ],
        out_specs=pl.BlockSpec((TBLK, ROW), lambda i: (i, 0)),
        out_shape=jax.ShapeDtypeStruct((HPAD, ROW), jnp.float32),
    )(tableT, tableT)


def _pool_body(ids_hbm, table_hbm, out_hbm, idx_v, gbuf, pooled_v, sems):
    wid = lax.axis_index("c") * NUM_SUBCORES + lax.axis_index("s")
    base = wid * B_PER_W
    pltpu.sync_copy(ids_hbm.at[pl.ds(base, B_PER_W), :], idx_v)

    inv_seq = jnp.float32(1.0 / SEQ)

    def start_gather(r, b):
        pltpu.async_copy(
            table_hbm.at[idx_v.at[r, pl.ds(0, S0)]],
            gbuf.at[b, pl.ds(0, S0)], sems.at[b])
        pltpu.async_copy(
            table_hbm.at[idx_v.at[r, pl.ds(S0, S1)]],
            gbuf.at[b, pl.ds(S0, S1)], sems.at[b])

    def wait_gather(b):
        pltpu.make_async_copy(
            table_hbm.at[idx_v.at[0, pl.ds(0, S0)]],
            gbuf.at[b, pl.ds(0, S0)], sems.at[b]).wait()
        pltpu.make_async_copy(
            table_hbm.at[idx_v.at[0, pl.ds(S0, S1)]],
            gbuf.at[b, pl.ds(S0, S1)], sems.at[b]).wait()

    def reduce_slot(r, b):
        def red_body(j, accs):
            accs = list(accs)
            for u in range(UNROLL):
                row = j * UNROLL + u
                for k in range(4):
                    a = u % 2 + 2 * k
                    accs[a] = accs[a] + gbuf[b, row, pl.ds(16 * k, 16)]
            return tuple(accs)

        zero = jnp.zeros((16,), jnp.float32)
        accs = lax.fori_loop(0, SEQ // UNROLL, red_body, (zero,) * 8)
        for k in range(4):
            pooled_v[r, pl.ds(16 * k, 16)] = (
                (accs[2 * k] + accs[2 * k + 1]) * inv_seq)

    for b in range(NBUF):
        start_gather(b, b)

    def outer(g, carry):
        for b in range(NBUF):
            r = g * NBUF + b
            wait_gather(b)
            reduce_slot(r, b)

            @pl.when(r + NBUF < B_PER_W)
            def _():
                start_gather(r + NBUF, b)
        return carry

    lax.fori_loop(0, B_PER_W // NBUF, outer, 0)
    pltpu.sync_copy(pooled_v, out_hbm.at[pl.ds(base, B_PER_W), :])


def _sc_pool(input_ids, table128):
    mesh = plsc.VectorSubcoreMesh(core_axis_name="c", subcore_axis_name="s")
    f = pl.kernel(
        _pool_body,
        out_type=jax.ShapeDtypeStruct((BATCH, EMBED_DIM), jnp.float32),
        mesh=mesh,
        scratch_types=[
            pltpu.VMEM((B_PER_W, SEQ), jnp.int32),
            pltpu.VMEM((NBUF, SEQ, EMBED_DIM), jnp.float32),
            pltpu.VMEM((B_PER_W, EMBED_DIM), jnp.float32),
            pltpu.SemaphoreType.DMA((NBUF,)),
        ],
        compiler_params=pltpu.CompilerParams(use_tc_tiling_on_sc=False),
    )
    return f(input_ids, table128)


BM = 256


def _matmul_body(p_ref, w_ref, b_ref, o_ref):
    acc = lax.dot_general(
        p_ref[...], w_ref[...],
        dimension_numbers=(((1,), (1,)), ((), ())),
        preferred_element_type=jnp.float32)
    o_ref[...] = acc + b_ref[...]


def _tc_head(pooled, fc_w, fc_b):
    bias = fc_b.reshape(1, NUM_CLASS)
    return pl.pallas_call(
        _matmul_body,
        grid=(BATCH // BM,),
        in_specs=[
            pl.BlockSpec((BM, EMBED_DIM), lambda i: (i, 0)),
            pl.BlockSpec((NUM_CLASS, EMBED_DIM), lambda i: (0, 0)),
            pl.BlockSpec((1, NUM_CLASS), lambda i: (0, 0)),
        ],
        out_specs=pl.BlockSpec((BM, NUM_CLASS), lambda i: (i, 0)),
        out_shape=jax.ShapeDtypeStruct((BATCH, NUM_CLASS), jnp.float32),
    )(pooled, fc_w, bias)


def kernel(input_ids, emb_table, fc_w, fc_b):
    tableT = emb_table.T
    table_pairs = _tc_transpose(tableT)
    table_lin = table_pairs.reshape(2 * HPAD, EMBED_DIM)
    ids2 = jnp.where(input_ids < HPAD,
                     2 * input_ids, 2 * (input_ids - HPAD) + 1)
    pooled = _sc_pool(ids2, table_lin)
    return _tc_head(pooled, fc_w, fc_b)
